# baseline (speedup 1.0000x reference)
import functools

import jax
import jax.numpy as jnp
from jax import lax
from jax.experimental import pallas as pl
from jax.experimental.pallas import tpu as pltpu
from jax.experimental.pallas import tpu_sc as plsc

E = 4096
C = 64
T = 16
NN = E // T
F = 2048
H = 4
B = 1024

FW = F // 32
NC, NS, L = 2, 16, 16
NW = NC * NS
CPW = C // NW
CB = 128
NCHK = B // CB
EPS = E // NS


def _sc_body(samples, tm, hm, filters, out, spT,
             stage1, stage2, chunkbuf, fpacked, tmbuf, hmbuf,
             lutL01, lutH01, lutL23, lutH23, rowbuf, resp):
    cid = lax.axis_index("c")
    sid = lax.axis_index("s")
    wid = sid * NC + cid
    c0 = wid * CPW
    iota = lax.iota(jnp.int32, L)
    iota32 = iota * 32

    def phase1(chunk, carry):
        for eg in range(8):
            pltpu.sync_copy(
                samples.at[pl.ds(chunk * CB, CB),
                           pl.ds(sid * EPS + eg * 32, 32)], stage1)

            def el_body(el, c2):
                colv = jnp.full((L,), el, jnp.int32)
                word = jnp.zeros((L,), jnp.int32)
                for p in range(8):
                    bits = plsc.load_gather(stage1, [p * 16 + iota, colv])
                    word = word | jnp.left_shift(bits, 4 * p)
                rowbuf[pl.ds(el * 64, 64)] = plsc.bitcast(word, jnp.int8)
                return c2

            lax.fori_loop(0, 32, el_body, 0)
            pltpu.sync_copy(
                rowbuf,
                spT.at[pl.ds(chunk * (E * 64) + sid * (EPS * 64) + eg * 2048,
                             2048)])
        return carry

    lax.fori_loop(0, NCHK, phase1, 0)

    def phase2(it, carry):
        cl = it >> 5
        ch = it & 31
        n0 = ch * 8
        pltpu.sync_copy(filters.at[c0 + cl, pl.ds(n0, 8)], stage2)

        def n_body(n, carry2):
            nv = jnp.full((L,), n, jnp.int32)
            fbase = (cl * NN + n0 + n) * FW

            def wv_body(wv, carry3):
                word = jnp.zeros((L,), jnp.int32)
                colb = iota32 + wv * (32 * L)
                for i in range(32):
                    bits = plsc.load_gather(stage2, [nv, colb + i])
                    word = word | jnp.left_shift(bits, i)
                fpacked[pl.ds(fbase + wv * L, L)] = word
                return carry3

            lax.fori_loop(0, FW // L, wv_body, 0)
            return carry2

        lax.fori_loop(0, 8, n_body, 0)
        return carry

    lax.fori_loop(0, CPW * 32, phase2, 0)

    pltpu.sync_copy(hm, hmbuf)
    hj01 = hmbuf[0, :] | jnp.left_shift(hmbuf[1, :], 16)
    hj23 = hmbuf[2, :] | jnp.left_shift(hmbuf[3, :], 16)
    for ref, hjv, jb in ((lutL01, hj01, 0), (lutH01, hj01, 8),
                         (lutL23, hj23, 0), (lutH23, hj23, 8)):
        hj = [hjv[jb + j] for j in range(8)]

        def xv_body(xv, carry, hj=hj, ref=ref):
            x = xv * L + iota
            a = jnp.zeros((L,), jnp.int32)
            for j in range(8):
                bit = jnp.right_shift(x, j) & 1
                a = a ^ ((0 - bit) & hj[j])
            ref[pl.ds(xv * L, L)] = a
            return carry

        lax.fori_loop(0, 256 // L, xv_body, 0)

    plsc.subcore_barrier()

    def half_members(loX, hiX, fb):
        h01 = (plsc.load_gather(lutL01, [loX])
               ^ plsc.load_gather(lutH01, [hiX]))
        h23 = (plsc.load_gather(lutL23, [loX])
               ^ plsc.load_gather(lutH23, [hiX]))
        m = None
        for h2p in (h01, h23):
            w0 = plsc.load_gather(
                fpacked, [fb + (jnp.right_shift(h2p, 5) & 0x3F)])
            b0 = jnp.right_shift(w0, h2p & 31)
            w1 = plsc.load_gather(fpacked, [fb + jnp.right_shift(h2p, 21)])
            b1 = jnp.right_shift(w1, jnp.right_shift(h2p, 16) & 31)
            mb = b0 & b1
            m = mb if m is None else (m & mb)
        return m & 1

    c11 = jnp.full((64,), 0x11, jnp.int8)
    MLO = 0x0F0F0F0F
    MHI = -252645136

    def phase4(chunk, carry):
        pltpu.sync_copy(spT.at[pl.ds(chunk * (E * 64), E * 64)], chunkbuf)
        for cl in range(CPW):
            pltpu.sync_copy(tm.at[pl.ds((c0 + cl) * E, E)], tmbuf)
            fb0 = cl * (NN * FW)

            def n_body(n, racc, fb0=fb0):
                tmv = tmbuf[pl.ds(n * T, T)]
                fb = jnp.full((L,), fb0 + n * FW, jnp.int32)
                accs = []
                for q in range(4):
                    a = jnp.zeros((64,), jnp.int8)
                    for jj in (3, 2, 1, 0):
                        e = tmv[q * 4 + jj]
                        v = chunkbuf[pl.ds(jnp.left_shift(e, 6), 64)]
                        a = a + a + (v & c11)
                    accs.append(plsc.bitcast(a, jnp.int32))
                a0, a1, a2, a3 = accs
                loE = (a0 & MLO) | (jnp.left_shift(a1, 4) & MHI)
                loO = (jnp.right_shift(a0, 4) & MLO) | (a1 & MHI)
                hiE = (a2 & MLO) | (jnp.left_shift(a3, 4) & MHI)
                hiO = (jnp.right_shift(a2, 4) & MLO) | (a3 & MHI)
                outs = [None] * 8
                for j in range(4):
                    sh = 8 * j
                    lo = jnp.right_shift(loE, sh) & 0xFF
                    hi = jnp.right_shift(hiE, sh) & 0xFF
                    outs[2 * j] = racc[2 * j] + half_members(lo, hi, fb)
                    lo = jnp.right_shift(loO, sh) & 0xFF
                    hi = jnp.right_shift(hiO, sh) & 0xFF
                    outs[2 * j + 1] = (racc[2 * j + 1]
                                       + half_members(lo, hi, fb))
                return tuple(outs)

            z = jnp.zeros((L,), jnp.int32)
            racc = lax.fori_loop(0, NN, n_body, (z,) * 8)
            rbase = cl * B + chunk * CB
            for p in range(8):
                resp[pl.ds(rbase + p * L, L)] = racc[p]
        return carry

    lax.fori_loop(0, NCHK, phase4, 0)

    pltpu.sync_copy(resp, out.at[pl.ds(wid * (CPW * B), CPW * B)])


_sc_kernel = functools.partial(
    pl.kernel,
    out_type=(jax.ShapeDtypeStruct((C * B,), jnp.int32),
              jax.ShapeDtypeStruct((NCHK * E * 64,), jnp.int8)),
    mesh=plsc.VectorSubcoreMesh(
        core_axis_name="c", subcore_axis_name="s",
        num_cores=NC, num_subcores=NS),
    compiler_params=pltpu.CompilerParams(
        use_tc_tiling_on_sc=False, needs_layout_passes=False),
    scratch_types=[
        pltpu.VMEM((CB, 32), jnp.int32),
        pltpu.VMEM((8, F), jnp.int32),
        pltpu.VMEM((E * 64,), jnp.int8),
        pltpu.VMEM((CPW * NN * FW,), jnp.int32),
        pltpu.VMEM((E,), jnp.int32),
        pltpu.VMEM((H, T), jnp.int32),
        pltpu.VMEM((256,), jnp.int32),
        pltpu.VMEM((256,), jnp.int32),
        pltpu.VMEM((256,), jnp.int32),
        pltpu.VMEM((256,), jnp.int32),
        pltpu.VMEM((2048,), jnp.int8),
        pltpu.VMEM((CPW * B,), jnp.int32),
    ],
)(_sc_body)


def kernel(samples, tuple_mapping, hash_matrix, filters):
    resp, _ = _sc_kernel(samples, tuple_mapping.reshape(-1), hash_matrix,
                         filters)
    return resp.reshape(C, B).T


# speedup vs baseline: 1.6258x; 1.6258x over previous
import functools

import jax
import jax.numpy as jnp
from jax import lax
from jax.experimental import pallas as pl
from jax.experimental.pallas import tpu as pltpu
from jax.experimental.pallas import tpu_sc as plsc

E = 4096
C = 64
T = 16
NN = E // T
F = 2048
H = 4
B = 1024

FW = F // 32
NC, NS, L = 2, 16, 16
NW = NC * NS
CPW = C // NW
CB = 128
NCHK = B // CB
EPS = E // NS


def _pack_body(fref, oref):
    x = fref[0].astype(jnp.float32)
    k = lax.broadcasted_iota(jnp.int32, (F, FW), 0)
    w = lax.broadcasted_iota(jnp.int32, (F, FW), 1)
    d = k - w * 32
    p_lo = jnp.where((d >= 0) & (d < 16),
                     jnp.left_shift(1, jnp.clip(d, 0, 15)), 0)
    p_hi = jnp.where((d >= 16) & (d < 32),
                     jnp.left_shift(1, jnp.clip(d - 16, 0, 15)), 0)
    lo = jnp.dot(x, p_lo.astype(jnp.float32),
                 preferred_element_type=jnp.float32).astype(jnp.int32)
    hi = jnp.dot(x, p_hi.astype(jnp.float32),
                 preferred_element_type=jnp.float32).astype(jnp.int32)
    oref[0] = lo | jnp.left_shift(hi, 16)


_pack_filters = pl.pallas_call(
    _pack_body,
    grid=(C,),
    in_specs=[pl.BlockSpec((1, NN, F), lambda c: (c, 0, 0))],
    out_specs=pl.BlockSpec((1, NN, FW), lambda c: (c, 0, 0)),
    out_shape=jax.ShapeDtypeStruct((C, NN, FW), jnp.int32),
)


def _sc_body(samples, tm, hm, fpk, out, spT,
             stage1, chunkbuf, fpacked, tmbuf, hmbuf,
             lutL01, lutH01, lutL23, lutH23, rowbuf, resp):
    cid = lax.axis_index("c")
    sid = lax.axis_index("s")
    wid = sid * NC + cid
    c0 = wid * CPW
    iota = lax.iota(jnp.int32, L)
    iota32 = iota * 32

    def phase1(chunk, carry):
        for eg in range(8):
            pltpu.sync_copy(
                samples.at[pl.ds(chunk * CB, CB),
                           pl.ds(sid * EPS + eg * 32, 32)], stage1)

            def el_body(el, c2):
                colv = jnp.full((L,), el, jnp.int32)
                word = jnp.zeros((L,), jnp.int32)
                for p in range(8):
                    bits = plsc.load_gather(stage1, [p * 16 + iota, colv])
                    word = word | jnp.left_shift(bits, 4 * p)
                rowbuf[pl.ds(el * 64, 64)] = plsc.bitcast(word, jnp.int8)
                return c2

            lax.fori_loop(0, 32, el_body, 0)
            pltpu.sync_copy(
                rowbuf,
                spT.at[pl.ds(chunk * (E * 64) + sid * (EPS * 64) + eg * 2048,
                             2048)])
        return carry

    lax.fori_loop(0, NCHK, phase1, 0)

    for cl in range(CPW):
        pltpu.sync_copy(fpk.at[c0 + cl],
                        fpacked.at[pl.ds(cl * (NN * FW), NN * FW)])

    pltpu.sync_copy(hm, hmbuf)
    hj01 = hmbuf[0, :] | jnp.left_shift(hmbuf[1, :], 16)
    hj23 = hmbuf[2, :] | jnp.left_shift(hmbuf[3, :], 16)
    for ref, hjv, jb in ((lutL01, hj01, 0), (lutH01, hj01, 8),
                         (lutL23, hj23, 0), (lutH23, hj23, 8)):
        hj = [hjv[jb + j] for j in range(8)]

        def xv_body(xv, carry, hj=hj, ref=ref):
            x = xv * L + iota
            a = jnp.zeros((L,), jnp.int32)
            for j in range(8):
                bit = jnp.right_shift(x, j) & 1
                a = a ^ ((0 - bit) & hj[j])
            ref[pl.ds(xv * L, L)] = a
            return carry

        lax.fori_loop(0, 256 // L, xv_body, 0)

    plsc.subcore_barrier()

    def half_members(loX, hiX, fb):
        h01 = (plsc.load_gather(lutL01, [loX])
               ^ plsc.load_gather(lutH01, [hiX]))
        h23 = (plsc.load_gather(lutL23, [loX])
               ^ plsc.load_gather(lutH23, [hiX]))
        m = None
        for h2p in (h01, h23):
            w0 = plsc.load_gather(
                fpacked, [fb + (jnp.right_shift(h2p, 5) & 0x3F)])
            b0 = jnp.right_shift(w0, h2p & 31)
            w1 = plsc.load_gather(fpacked, [fb + jnp.right_shift(h2p, 21)])
            b1 = jnp.right_shift(w1, jnp.right_shift(h2p, 16) & 31)
            mb = b0 & b1
            m = mb if m is None else (m & mb)
        return m & 1

    c11 = jnp.full((64,), 0x11, jnp.int8)
    MLO = 0x0F0F0F0F
    MHI = -252645136

    def phase4(chunk, carry):
        pltpu.sync_copy(spT.at[pl.ds(chunk * (E * 64), E * 64)], chunkbuf)
        for cl in range(CPW):
            pltpu.sync_copy(tm.at[pl.ds((c0 + cl) * E, E)], tmbuf)
            fb0 = cl * (NN * FW)

            def n_body(n, racc, fb0=fb0):
                tmv = tmbuf[pl.ds(n * T, T)]
                fb = jnp.full((L,), fb0 + n * FW, jnp.int32)
                accs = []
                for q in range(4):
                    a = jnp.zeros((64,), jnp.int8)
                    for jj in (3, 2, 1, 0):
                        e = tmv[q * 4 + jj]
                        v = chunkbuf[pl.ds(jnp.left_shift(e, 6), 64)]
                        a = a + a + (v & c11)
                    accs.append(plsc.bitcast(a, jnp.int32))
                a0, a1, a2, a3 = accs
                loE = (a0 & MLO) | (jnp.left_shift(a1, 4) & MHI)
                loO = (jnp.right_shift(a0, 4) & MLO) | (a1 & MHI)
                hiE = (a2 & MLO) | (jnp.left_shift(a3, 4) & MHI)
                hiO = (jnp.right_shift(a2, 4) & MLO) | (a3 & MHI)
                outs = [None] * 8
                for j in range(4):
                    sh = 8 * j
                    lo = jnp.right_shift(loE, sh) & 0xFF
                    hi = jnp.right_shift(hiE, sh) & 0xFF
                    outs[2 * j] = racc[2 * j] + half_members(lo, hi, fb)
                    lo = jnp.right_shift(loO, sh) & 0xFF
                    hi = jnp.right_shift(hiO, sh) & 0xFF
                    outs[2 * j + 1] = (racc[2 * j + 1]
                                       + half_members(lo, hi, fb))
                return tuple(outs)

            z = jnp.zeros((L,), jnp.int32)
            racc = lax.fori_loop(0, NN, n_body, (z,) * 8)
            rbase = cl * B + chunk * CB
            for p in range(8):
                resp[pl.ds(rbase + p * L, L)] = racc[p]
        return carry

    lax.fori_loop(0, NCHK, phase4, 0)

    pltpu.sync_copy(resp, out.at[pl.ds(wid * (CPW * B), CPW * B)])


_sc_kernel = functools.partial(
    pl.kernel,
    out_type=(jax.ShapeDtypeStruct((C * B,), jnp.int32),
              jax.ShapeDtypeStruct((NCHK * E * 64,), jnp.int8)),
    mesh=plsc.VectorSubcoreMesh(
        core_axis_name="c", subcore_axis_name="s",
        num_cores=NC, num_subcores=NS),
    compiler_params=pltpu.CompilerParams(
        use_tc_tiling_on_sc=False, needs_layout_passes=False),
    scratch_types=[
        pltpu.VMEM((CB, 32), jnp.int32),
        pltpu.VMEM((E * 64,), jnp.int8),
        pltpu.VMEM((CPW * NN * FW,), jnp.int32),
        pltpu.VMEM((E,), jnp.int32),
        pltpu.VMEM((H, T), jnp.int32),
        pltpu.VMEM((256,), jnp.int32),
        pltpu.VMEM((256,), jnp.int32),
        pltpu.VMEM((256,), jnp.int32),
        pltpu.VMEM((256,), jnp.int32),
        pltpu.VMEM((2048,), jnp.int8),
        pltpu.VMEM((CPW * B,), jnp.int32),
    ],
)(_sc_body)


def kernel(samples, tuple_mapping, hash_matrix, filters):
    fpk = _pack_filters(filters).reshape(C, NN * FW)
    resp, _ = _sc_kernel(samples, tuple_mapping.reshape(-1), hash_matrix,
                         fpk)
    return resp.reshape(C, B).T


# speedup vs baseline: 2.1305x; 1.3104x over previous
import functools

import jax
import jax.numpy as jnp
from jax import lax
from jax.experimental import pallas as pl
from jax.experimental.pallas import tpu as pltpu
from jax.experimental.pallas import tpu_sc as plsc

E = 4096
C = 64
T = 16
NN = E // T
F = 2048
H = 4
B = 1024

FW = F // 32
NC, NS, L = 2, 16, 16
NW = NC * NS
CPW = C // NW
CB = 128
NCHK = B // CB
EPS = E // NS


def _pack_body(fref, oref):
    x = fref[0].astype(jnp.float32)
    k = lax.broadcasted_iota(jnp.int32, (F, FW), 0)
    w = lax.broadcasted_iota(jnp.int32, (F, FW), 1)
    d = k - w * 32
    p_lo = jnp.where((d >= 0) & (d < 16),
                     jnp.left_shift(1, jnp.clip(d, 0, 15)), 0)
    p_hi = jnp.where((d >= 16) & (d < 32),
                     jnp.left_shift(1, jnp.clip(d - 16, 0, 15)), 0)
    lo = jnp.dot(x, p_lo.astype(jnp.float32),
                 preferred_element_type=jnp.float32).astype(jnp.int32)
    hi = jnp.dot(x, p_hi.astype(jnp.float32),
                 preferred_element_type=jnp.float32).astype(jnp.int32)
    oref[0] = lo | jnp.left_shift(hi, 16)


_pack_filters = pl.pallas_call(
    _pack_body,
    grid=(C,),
    in_specs=[pl.BlockSpec((1, NN, F), lambda c: (c, 0, 0))],
    out_specs=pl.BlockSpec((1, NN, FW), lambda c: (c, 0, 0)),
    out_shape=jax.ShapeDtypeStruct((C, NN, FW), jnp.int32),
)


def _spack_body(sref, oref):
    x = sref[...].astype(jnp.float32)
    k = lax.broadcasted_iota(jnp.int32, (CB, L), 0)
    lidx = lax.broadcasted_iota(jnp.int32, (CB, L), 1)
    p = jnp.right_shift(k, 4)
    match = (k & 15) == lidx
    wlo = jnp.where(match & (p < 4),
                    jnp.left_shift(1, 4 * jnp.clip(p, 0, 3)),
                    0).astype(jnp.float32)
    whi = jnp.where(match & (p >= 4),
                    jnp.left_shift(1, 4 * jnp.clip(p - 4, 0, 3)),
                    0).astype(jnp.float32)
    dn = (((0,), (0,)), ((), ()))
    lo = lax.dot_general(x, wlo, dn,
                         preferred_element_type=jnp.float32).astype(jnp.int32)
    hi = lax.dot_general(x, whi, dn,
                         preferred_element_type=jnp.float32).astype(jnp.int32)
    oref[0] = lo | jnp.left_shift(hi, 16)


_pack_samples = pl.pallas_call(
    _spack_body,
    grid=(NCHK,),
    in_specs=[pl.BlockSpec((CB, E), lambda c: (c, 0))],
    out_specs=pl.BlockSpec((1, E, L), lambda c: (c, 0, 0)),
    out_shape=jax.ShapeDtypeStruct((NCHK, E, L), jnp.int32),
)


def _sc_body(spT, tm, hm, fpk, out,
             chunkbuf, fpacked, tmbuf, hmbuf,
             lutL01, lutH01, lutL23, lutH23, resp):
    cid = lax.axis_index("c")
    sid = lax.axis_index("s")
    wid = sid * NC + cid
    c0 = wid * CPW
    iota = lax.iota(jnp.int32, L)

    for cl in range(CPW):
        pltpu.sync_copy(fpk.at[c0 + cl],
                        fpacked.at[pl.ds(cl * (NN * FW), NN * FW)])

    pltpu.sync_copy(hm, hmbuf)
    hj01 = hmbuf[0, :] | jnp.left_shift(hmbuf[1, :], 16)
    hj23 = hmbuf[2, :] | jnp.left_shift(hmbuf[3, :], 16)
    for ref, hjv, jb in ((lutL01, hj01, 0), (lutH01, hj01, 8),
                         (lutL23, hj23, 0), (lutH23, hj23, 8)):
        hj = [hjv[jb + j] for j in range(8)]

        def xv_body(xv, carry, hj=hj, ref=ref):
            x = xv * L + iota
            a = jnp.zeros((L,), jnp.int32)
            for j in range(8):
                bit = jnp.right_shift(x, j) & 1
                a = a ^ ((0 - bit) & hj[j])
            ref[pl.ds(xv * L, L)] = a
            return carry

        lax.fori_loop(0, 256 // L, xv_body, 0)

    def half_members(loX, hiX, fb):
        h01 = (plsc.load_gather(lutL01, [loX])
               ^ plsc.load_gather(lutH01, [hiX]))
        h23 = (plsc.load_gather(lutL23, [loX])
               ^ plsc.load_gather(lutH23, [hiX]))
        m = None
        for h2p in (h01, h23):
            w0 = plsc.load_gather(
                fpacked, [fb + (jnp.right_shift(h2p, 5) & 0x3F)])
            b0 = jnp.right_shift(w0, h2p & 31)
            w1 = plsc.load_gather(fpacked, [fb + jnp.right_shift(h2p, 21)])
            b1 = jnp.right_shift(w1, jnp.right_shift(h2p, 16) & 31)
            mb = b0 & b1
            m = mb if m is None else (m & mb)
        return m & 1

    c11 = jnp.full((64,), 0x11, jnp.int8)
    MLO = 0x0F0F0F0F
    MHI = -252645136

    def phase4(chunk, carry):
        pltpu.sync_copy(spT.at[pl.ds(chunk * (E * L), E * L)], chunkbuf)
        for cl in range(CPW):
            pltpu.sync_copy(tm.at[pl.ds((c0 + cl) * E, E)], tmbuf)
            fb0 = cl * (NN * FW)

            def n_body(n, racc, fb0=fb0):
                tmv = tmbuf[pl.ds(n * T, T)]
                fb = jnp.full((L,), fb0 + n * FW, jnp.int32)
                accs = []
                for q in range(4):
                    a = jnp.zeros((64,), jnp.int8)
                    for jj in (3, 2, 1, 0):
                        e = tmv[q * 4 + jj]
                        v = plsc.bitcast(
                            chunkbuf[pl.ds(jnp.left_shift(e, 4), L)],
                            jnp.int8)
                        a = a + a + (v & c11)
                    accs.append(plsc.bitcast(a, jnp.int32))
                a0, a1, a2, a3 = accs
                loE = (a0 & MLO) | (jnp.left_shift(a1, 4) & MHI)
                loO = (jnp.right_shift(a0, 4) & MLO) | (a1 & MHI)
                hiE = (a2 & MLO) | (jnp.left_shift(a3, 4) & MHI)
                hiO = (jnp.right_shift(a2, 4) & MLO) | (a3 & MHI)
                outs = [None] * 8
                for j in range(4):
                    sh = 8 * j
                    lo = jnp.right_shift(loE, sh) & 0xFF
                    hi = jnp.right_shift(hiE, sh) & 0xFF
                    outs[2 * j] = racc[2 * j] + half_members(lo, hi, fb)
                    lo = jnp.right_shift(loO, sh) & 0xFF
                    hi = jnp.right_shift(hiO, sh) & 0xFF
                    outs[2 * j + 1] = (racc[2 * j + 1]
                                       + half_members(lo, hi, fb))
                return tuple(outs)

            z = jnp.zeros((L,), jnp.int32)
            racc = lax.fori_loop(0, NN, n_body, (z,) * 8)
            rbase = cl * B + chunk * CB
            for p in range(8):
                resp[pl.ds(rbase + p * L, L)] = racc[p]
        return carry

    lax.fori_loop(0, NCHK, phase4, 0)

    pltpu.sync_copy(resp, out.at[pl.ds(wid * (CPW * B), CPW * B)])


_sc_kernel = functools.partial(
    pl.kernel,
    out_type=jax.ShapeDtypeStruct((C * B,), jnp.int32),
    mesh=plsc.VectorSubcoreMesh(
        core_axis_name="c", subcore_axis_name="s",
        num_cores=NC, num_subcores=NS),
    compiler_params=pltpu.CompilerParams(
        use_tc_tiling_on_sc=False, needs_layout_passes=False),
    scratch_types=[
        pltpu.VMEM((E * L,), jnp.int32),
        pltpu.VMEM((CPW * NN * FW,), jnp.int32),
        pltpu.VMEM((E,), jnp.int32),
        pltpu.VMEM((H, T), jnp.int32),
        pltpu.VMEM((256,), jnp.int32),
        pltpu.VMEM((256,), jnp.int32),
        pltpu.VMEM((256,), jnp.int32),
        pltpu.VMEM((256,), jnp.int32),
        pltpu.VMEM((CPW * B,), jnp.int32),
    ],
)(_sc_body)


def kernel(samples, tuple_mapping, hash_matrix, filters):
    fpk = _pack_filters(filters).reshape(C, NN * FW)
    spT = _pack_samples(samples).reshape(-1)
    resp = _sc_kernel(spT, tuple_mapping.reshape(-1), hash_matrix, fpk)
    return resp.reshape(C, B).T
